# Initial kernel scaffold; baseline (speedup 1.0000x reference)
import jax
import jax.numpy as jnp
from jax.experimental import pallas as pl


def kernel(x, coefficients):
    raise NotImplementedError("write your pallas kernel here")


# speedup vs baseline: 1557.8049x; 1557.8049x over previous
import functools

import numpy as np
import jax
import jax.numpy as jnp
from jax import lax
from jax.experimental import pallas as pl
from jax.experimental.pallas import tpu as pltpu
from jax.experimental.pallas import tpu_sc as plsc

_NUM_ACT = 96
_NUM_KNOTS = 64
_X_MIN = -4.0
_X_MAX = 4.0
_STEP = (_X_MAX - _X_MIN) / (_NUM_KNOTS - 1)
_INV_STEP = (_NUM_KNOTS - 1) / (_X_MAX - _X_MIN)
_TMAX = float(_NUM_KNOTS - 2)
_XS_TOP = float(np.float32(np.float32(_X_MAX - _STEP) - np.float32(_X_MIN)))

_NC, _NS, _L = 2, 16, 16
_NW = _NC * _NS

_B, _C, _H, _W = 4, 96, 384, 384
_ROWS = _B * _C
_ROW_LEN = _H * _W
_ROWS_PER = _ROWS // _NW
_CHUNK = 16384
_NCHUNK = _ROW_LEN // _CHUNK
_UNROLL = 4


def _project_body(cs_ref, c_ref, d_ref):
    cs = cs_ref[...]
    i2 = lax.broadcasted_iota(jnp.int32, (_NUM_KNOTS, _NUM_KNOTS), 0)
    j2 = lax.broadcasted_iota(jnp.int32, (_NUM_KNOTS, _NUM_KNOTS), 1)
    col = lax.broadcasted_iota(jnp.int32, (_NUM_ACT, _NUM_KNOTS), 1)

    hi = jax.lax.Precision.HIGHEST
    m_prev = (i2 == (j2 - 1)).astype(jnp.float32)
    prev = jnp.dot(cs, m_prev, precision=hi, preferred_element_type=jnp.float32)
    s = (cs - prev) * jnp.float32(1.0 / _STEP)
    s = jnp.where((col >= 2) & (col <= _NUM_KNOTS - 2), s, 0.0)
    tri = (i2 <= j2).astype(jnp.float32)
    new_cs = jnp.dot(s, tri, precision=hi, preferred_element_type=jnp.float32) * jnp.float32(_STEP)
    adj = jnp.mean(cs - new_cs, axis=1, keepdims=True)
    c = new_cs + adj
    m_next = (i2 == (j2 + 1)).astype(jnp.float32)
    nxt = jnp.dot(c, m_next, precision=hi, preferred_element_type=jnp.float32)
    d = jnp.where(col <= _NUM_KNOTS - 2, nxt - c, 0.0)
    c_ref[...] = c
    d_ref[...] = d


def _project_tables(coefficients):
    return pl.pallas_call(
        _project_body,
        out_shape=[
            jax.ShapeDtypeStruct((_NUM_ACT, _NUM_KNOTS), jnp.float32),
            jax.ShapeDtypeStruct((_NUM_ACT, _NUM_KNOTS), jnp.float32),
        ],
    )(coefficients)


def _sc_body(x_hbm, c_hbm, d_hbm, out_hbm,
             in0, in1, out0, out1, crow, drow,
             sin0, sin1, sout0, sout1):
    wid = lax.axis_index("s") * _NC + lax.axis_index("c")
    inbufs = (in0, in1)
    outbufs = (out0, out1)
    sins = (sin0, sin1)
    souts = (sout0, sout1)

    def do_row(r, carry):
        row = wid * _ROWS_PER + r
        chan = lax.rem(row, _NUM_ACT)
        pltpu.sync_copy(c_hbm.at[chan], crow)
        pltpu.sync_copy(d_hbm.at[chan], drow)

        hin = [None, None]
        hout = [None, None]
        hin[0] = pltpu.async_copy(
            x_hbm.at[row, pl.ds(0, _CHUNK)], inbufs[0], sins[0])
        for k in range(_NCHUNK):
            b = k % 2
            if k + 1 < _NCHUNK:
                nb = (k + 1) % 2
                hin[nb] = pltpu.async_copy(
                    x_hbm.at[row, pl.ds((k + 1) * _CHUNK, _CHUNK)],
                    inbufs[nb], sins[nb])
            hin[b].wait()
            if k >= 2:
                hout[b].wait()
            ib = inbufs[b]
            ob = outbufs[b]

            @plsc.parallel_loop(0, _CHUNK, step=_L * _UNROLL, unroll=1)
            def _compute(i):
                for u in range(_UNROLL):
                    off = i + u * _L
                    xv = ib[pl.ds(off, _L)]
                    xs = xv - jnp.float32(_X_MIN)
                    xsc = jnp.minimum(jnp.maximum(xs, 0.0), jnp.float32(_XS_TOP))
                    idx = (xsc / jnp.float32(_STEP)).astype(jnp.int32)
                    fr = xs * jnp.float32(_INV_STEP) - idx.astype(jnp.float32)
                    c0 = plsc.load_gather(crow, [idx])
                    dd = plsc.load_gather(drow, [idx])
                    ob[pl.ds(off, _L)] = c0 + dd * fr

            hout[b] = pltpu.async_copy(
                ob, out_hbm.at[row, pl.ds(k * _CHUNK, _CHUNK)], souts[b])
        hout[0].wait()
        hout[1].wait()
        return carry

    lax.fori_loop(0, _ROWS_PER, do_row, 0)


@jax.jit
def kernel(x, coefficients):
    ctab, dtab = _project_tables(coefficients)
    xf = x.reshape(_ROWS, _ROW_LEN)

    mesh = plsc.VectorSubcoreMesh(core_axis_name="c", subcore_axis_name="s")
    run = pl.kernel(
        _sc_body,
        out_type=jax.ShapeDtypeStruct((_ROWS, _ROW_LEN), jnp.float32),
        mesh=mesh,
        compiler_params=pltpu.CompilerParams(needs_layout_passes=False),
        scratch_types=[
            pltpu.VMEM((_CHUNK,), jnp.float32),
            pltpu.VMEM((_CHUNK,), jnp.float32),
            pltpu.VMEM((_CHUNK,), jnp.float32),
            pltpu.VMEM((_CHUNK,), jnp.float32),
            pltpu.VMEM((_NUM_KNOTS,), jnp.float32),
            pltpu.VMEM((_NUM_KNOTS,), jnp.float32),
            pltpu.SemaphoreType.DMA,
            pltpu.SemaphoreType.DMA,
            pltpu.SemaphoreType.DMA,
            pltpu.SemaphoreType.DMA,
        ],
    )
    out = run(xf, ctab, dtab)
    return out.reshape(x.shape)


# speedup vs baseline: 1595.6153x; 1.0243x over previous
import functools

import numpy as np
import jax
import jax.numpy as jnp
from jax import lax
from jax.experimental import pallas as pl
from jax.experimental.pallas import tpu as pltpu
from jax.experimental.pallas import tpu_sc as plsc

_NUM_ACT = 96
_NUM_KNOTS = 64
_X_MIN = -4.0
_X_MAX = 4.0
_STEP = (_X_MAX - _X_MIN) / (_NUM_KNOTS - 1)
_INV_STEP = (_NUM_KNOTS - 1) / (_X_MAX - _X_MIN)
_TMAX = float(_NUM_KNOTS - 2)
_XS_TOP = float(np.float32(np.float32(_X_MAX - _STEP) - np.float32(_X_MIN)))
_T_TOP = float(np.float32(np.float32(_XS_TOP) / np.float32(_STEP)))

_NC, _NS, _L = 2, 16, 16
_NW = _NC * _NS

_B, _C, _H, _W = 4, 96, 384, 384
_ROWS = _B * _C
_ROW_LEN = _H * _W
_ROWS_PER = _ROWS // _NW
_CHUNK = 16384
_NCHUNK = _ROW_LEN // _CHUNK
_UNROLL = 4


def _project_body(cs_ref, c_ref, d_ref):
    cs = cs_ref[...]
    i2 = lax.broadcasted_iota(jnp.int32, (_NUM_KNOTS, _NUM_KNOTS), 0)
    j2 = lax.broadcasted_iota(jnp.int32, (_NUM_KNOTS, _NUM_KNOTS), 1)
    col = lax.broadcasted_iota(jnp.int32, (_NUM_ACT, _NUM_KNOTS), 1)

    hi = jax.lax.Precision.HIGHEST
    m_prev = (i2 == (j2 - 1)).astype(jnp.float32)
    prev = jnp.dot(cs, m_prev, precision=hi, preferred_element_type=jnp.float32)
    s = (cs - prev) * jnp.float32(1.0 / _STEP)
    s = jnp.where((col >= 2) & (col <= _NUM_KNOTS - 2), s, 0.0)
    tri = (i2 <= j2).astype(jnp.float32)
    new_cs = jnp.dot(s, tri, precision=hi, preferred_element_type=jnp.float32) * jnp.float32(_STEP)
    adj = jnp.mean(cs - new_cs, axis=1, keepdims=True)
    c = new_cs + adj
    m_next = (i2 == (j2 + 1)).astype(jnp.float32)
    nxt = jnp.dot(c, m_next, precision=hi, preferred_element_type=jnp.float32)
    d = jnp.where(col <= _NUM_KNOTS - 2, nxt - c, 0.0)
    c_ref[...] = c
    d_ref[...] = d


def _project_tables(coefficients):
    return pl.pallas_call(
        _project_body,
        out_shape=[
            jax.ShapeDtypeStruct((_NUM_ACT, _NUM_KNOTS), jnp.float32),
            jax.ShapeDtypeStruct((_NUM_ACT, _NUM_KNOTS), jnp.float32),
        ],
    )(coefficients)


def _sc_body(x_hbm, c_hbm, d_hbm, out_hbm,
             in0, in1, out0, out1, crow, drow,
             sin0, sin1, sout0, sout1):
    wid = lax.axis_index("s") * _NC + lax.axis_index("c")
    inbufs = (in0, in1)
    outbufs = (out0, out1)
    sins = (sin0, sin1)
    souts = (sout0, sout1)

    def do_row(r, carry):
        row = wid * _ROWS_PER + r
        chan = lax.rem(row, _NUM_ACT)
        pltpu.sync_copy(c_hbm.at[chan], crow)
        pltpu.sync_copy(d_hbm.at[chan], drow)

        hin = [None, None]
        hout = [None, None]
        hin[0] = pltpu.async_copy(
            x_hbm.at[row, pl.ds(0, _CHUNK)], inbufs[0], sins[0])
        for k in range(_NCHUNK):
            b = k % 2
            if k + 1 < _NCHUNK:
                nb = (k + 1) % 2
                hin[nb] = pltpu.async_copy(
                    x_hbm.at[row, pl.ds((k + 1) * _CHUNK, _CHUNK)],
                    inbufs[nb], sins[nb])
            hin[b].wait()
            if k >= 2:
                hout[b].wait()
            ib = inbufs[b]
            ob = outbufs[b]

            @plsc.parallel_loop(0, _CHUNK, step=_L * _UNROLL, unroll=1)
            def _compute(i):
                for u in range(_UNROLL):
                    off = i + u * _L
                    xv = ib[pl.ds(off, _L)]
                    t = (xv - jnp.float32(_X_MIN)) * jnp.float32(_INV_STEP)
                    tcl = jnp.minimum(jnp.maximum(t, 0.0), jnp.float32(_T_TOP))
                    idx = tcl.astype(jnp.int32)
                    fr = t - idx.astype(jnp.float32)
                    c0 = plsc.load_gather(crow, [idx])
                    dd = plsc.load_gather(drow, [idx])
                    ob[pl.ds(off, _L)] = c0 + dd * fr

            hout[b] = pltpu.async_copy(
                ob, out_hbm.at[row, pl.ds(k * _CHUNK, _CHUNK)], souts[b])
        hout[0].wait()
        hout[1].wait()
        return carry

    lax.fori_loop(0, _ROWS_PER, do_row, 0)


@jax.jit
def kernel(x, coefficients):
    ctab, dtab = _project_tables(coefficients)
    xf = x.reshape(_ROWS, _ROW_LEN)

    mesh = plsc.VectorSubcoreMesh(core_axis_name="c", subcore_axis_name="s")
    run = pl.kernel(
        _sc_body,
        out_type=jax.ShapeDtypeStruct((_ROWS, _ROW_LEN), jnp.float32),
        mesh=mesh,
        compiler_params=pltpu.CompilerParams(needs_layout_passes=False),
        scratch_types=[
            pltpu.VMEM((_CHUNK,), jnp.float32),
            pltpu.VMEM((_CHUNK,), jnp.float32),
            pltpu.VMEM((_CHUNK,), jnp.float32),
            pltpu.VMEM((_CHUNK,), jnp.float32),
            pltpu.VMEM((_NUM_KNOTS,), jnp.float32),
            pltpu.VMEM((_NUM_KNOTS,), jnp.float32),
            pltpu.SemaphoreType.DMA,
            pltpu.SemaphoreType.DMA,
            pltpu.SemaphoreType.DMA,
            pltpu.SemaphoreType.DMA,
        ],
    )
    out = run(xf, ctab, dtab)
    return out.reshape(x.shape)


# speedup vs baseline: 3017.0329x; 1.8908x over previous
import functools

import numpy as np
import jax
import jax.numpy as jnp
from jax import lax
from jax.experimental import pallas as pl
from jax.experimental.pallas import tpu as pltpu
from jax.experimental.pallas import tpu_sc as plsc

_NUM_ACT = 96
_NUM_KNOTS = 64
_X_MIN = -4.0
_X_MAX = 4.0
_STEP = (_X_MAX - _X_MIN) / (_NUM_KNOTS - 1)
_INV_STEP = (_NUM_KNOTS - 1) / (_X_MAX - _X_MIN)
_TMAX = float(_NUM_KNOTS - 2)
_XS_TOP = float(np.float32(np.float32(_X_MAX - _STEP) - np.float32(_X_MIN)))
_T_TOP = float(np.float32(np.float32(_XS_TOP) / np.float32(_STEP)))

_NC, _NS, _L = 2, 16, 16
_NW = _NC * _NS

_B, _C, _H, _W = 4, 96, 384, 384
_IMGS = _B * _C
_IMGS_PER = _IMGS // _NW
_HCHUNK = 48
_NCHUNK = _H // _HCHUNK
_WVECS = _W // _L


def _project_body(cs_ref, c_ref, d_ref):
    cs = cs_ref[...]
    i2 = lax.broadcasted_iota(jnp.int32, (_NUM_KNOTS, _NUM_KNOTS), 0)
    j2 = lax.broadcasted_iota(jnp.int32, (_NUM_KNOTS, _NUM_KNOTS), 1)
    col = lax.broadcasted_iota(jnp.int32, (_NUM_ACT, _NUM_KNOTS), 1)

    hi = jax.lax.Precision.HIGHEST
    m_prev = (i2 == (j2 - 1)).astype(jnp.float32)
    prev = jnp.dot(cs, m_prev, precision=hi, preferred_element_type=jnp.float32)
    s = (cs - prev) * jnp.float32(1.0 / _STEP)
    s = jnp.where((col >= 2) & (col <= _NUM_KNOTS - 2), s, 0.0)
    tri = (i2 <= j2).astype(jnp.float32)
    new_cs = jnp.dot(s, tri, precision=hi, preferred_element_type=jnp.float32) * jnp.float32(_STEP)
    adj = jnp.mean(cs - new_cs, axis=1, keepdims=True)
    c = new_cs + adj
    m_next = (i2 == (j2 + 1)).astype(jnp.float32)
    nxt = jnp.dot(c, m_next, precision=hi, preferred_element_type=jnp.float32)
    d = jnp.where(col <= _NUM_KNOTS - 2, nxt - c, 0.0)
    c_ref[...] = c
    d_ref[...] = d


def _project_tables(coefficients):
    return pl.pallas_call(
        _project_body,
        out_shape=[
            jax.ShapeDtypeStruct((_NUM_ACT, _NUM_KNOTS), jnp.float32),
            jax.ShapeDtypeStruct((_NUM_ACT, _NUM_KNOTS), jnp.float32),
        ],
    )(coefficients)


def _sc_body(x_hbm, c_hbm, d_hbm, out_hbm,
             in0, in1, out0, out1, crow, drow,
             sin0, sin1, sout0, sout1):
    wid = lax.axis_index("s") * _NC + lax.axis_index("c")
    inbufs = (in0, in1)
    outbufs = (out0, out1)
    sins = (sin0, sin1)
    souts = (sout0, sout1)

    def do_img(r, carry):
        img = wid * _IMGS_PER + r
        chan = lax.rem(img, _NUM_ACT)
        pltpu.sync_copy(c_hbm.at[chan], crow)
        pltpu.sync_copy(d_hbm.at[chan], drow)

        hin = [None, None]
        hout = [None, None]
        hin[0] = pltpu.async_copy(
            x_hbm.at[img, pl.ds(0, _HCHUNK), :], inbufs[0], sins[0])
        for k in range(_NCHUNK):
            b = k % 2
            if k + 1 < _NCHUNK:
                nb = (k + 1) % 2
                hin[nb] = pltpu.async_copy(
                    x_hbm.at[img, pl.ds((k + 1) * _HCHUNK, _HCHUNK), :],
                    inbufs[nb], sins[nb])
            hin[b].wait()
            if k >= 2:
                hout[b].wait()
            ib = inbufs[b]
            ob = outbufs[b]

            @plsc.parallel_loop(0, _HCHUNK, step=1, unroll=1)
            def _compute(rr):
                for u in range(_WVECS):
                    off = u * _L
                    xv = ib[rr, pl.ds(off, _L)]
                    t = (xv - jnp.float32(_X_MIN)) * jnp.float32(_INV_STEP)
                    tcl = jnp.minimum(jnp.maximum(t, 0.0), jnp.float32(_T_TOP))
                    idx = tcl.astype(jnp.int32)
                    fr = t - idx.astype(jnp.float32)
                    c0 = plsc.load_gather(crow, [idx])
                    dd = plsc.load_gather(drow, [idx])
                    ob[rr, pl.ds(off, _L)] = c0 + dd * fr

            hout[b] = pltpu.async_copy(
                ob, out_hbm.at[img, pl.ds(k * _HCHUNK, _HCHUNK), :], souts[b])
        hout[0].wait()
        hout[1].wait()
        return carry

    lax.fori_loop(0, _IMGS_PER, do_img, 0)


@jax.jit
def kernel(x, coefficients):
    ctab, dtab = _project_tables(coefficients)
    xf = x.reshape(_IMGS, _H, _W)

    mesh = plsc.VectorSubcoreMesh(core_axis_name="c", subcore_axis_name="s")
    run = pl.kernel(
        _sc_body,
        out_type=jax.ShapeDtypeStruct((_IMGS, _H, _W), jnp.float32),
        mesh=mesh,
        compiler_params=pltpu.CompilerParams(needs_layout_passes=False),
        scratch_types=[
            pltpu.VMEM((_HCHUNK, _W), jnp.float32),
            pltpu.VMEM((_HCHUNK, _W), jnp.float32),
            pltpu.VMEM((_HCHUNK, _W), jnp.float32),
            pltpu.VMEM((_HCHUNK, _W), jnp.float32),
            pltpu.VMEM((_NUM_KNOTS,), jnp.float32),
            pltpu.VMEM((_NUM_KNOTS,), jnp.float32),
            pltpu.SemaphoreType.DMA,
            pltpu.SemaphoreType.DMA,
            pltpu.SemaphoreType.DMA,
            pltpu.SemaphoreType.DMA,
        ],
    )
    out = run(xf, ctab, dtab)
    return out.reshape(x.shape)


# speedup vs baseline: 3426.3779x; 1.1357x over previous
import functools

import numpy as np
import jax
import jax.numpy as jnp
from jax import lax
from jax.experimental import pallas as pl
from jax.experimental.pallas import tpu as pltpu
from jax.experimental.pallas import tpu_sc as plsc

_NUM_ACT = 96
_NUM_KNOTS = 64
_X_MIN = -4.0
_X_MAX = 4.0
_STEP = (_X_MAX - _X_MIN) / (_NUM_KNOTS - 1)
_INV_STEP = (_NUM_KNOTS - 1) / (_X_MAX - _X_MIN)
_TMAX = float(_NUM_KNOTS - 2)
_XS_TOP = float(np.float32(np.float32(_X_MAX - _STEP) - np.float32(_X_MIN)))
_T_TOP = float(np.float32(np.float32(_XS_TOP) / np.float32(_STEP)))

_NC, _NS, _L = 2, 16, 16
_NW = _NC * _NS

_B, _C, _H, _W = 4, 96, 384, 384
_IMGS = _B * _C
_IMGS_PER = _IMGS // _NW
_HCHUNK = 64
_NCHUNK = _H // _HCHUNK
_WVECS = _W // _L
_NTOT = _IMGS_PER * _NCHUNK


def _project_body(cs_ref, c_ref, d_ref):
    cs = cs_ref[...]
    i2 = lax.broadcasted_iota(jnp.int32, (_NUM_KNOTS, _NUM_KNOTS), 0)
    j2 = lax.broadcasted_iota(jnp.int32, (_NUM_KNOTS, _NUM_KNOTS), 1)
    col = lax.broadcasted_iota(jnp.int32, (_NUM_ACT, _NUM_KNOTS), 1)

    hi = jax.lax.Precision.HIGHEST
    m_prev = (i2 == (j2 - 1)).astype(jnp.float32)
    prev = jnp.dot(cs, m_prev, precision=hi, preferred_element_type=jnp.float32)
    s = (cs - prev) * jnp.float32(1.0 / _STEP)
    s = jnp.where((col >= 2) & (col <= _NUM_KNOTS - 2), s, 0.0)
    tri = (i2 <= j2).astype(jnp.float32)
    new_cs = jnp.dot(s, tri, precision=hi, preferred_element_type=jnp.float32) * jnp.float32(_STEP)
    adj = jnp.mean(cs - new_cs, axis=1, keepdims=True)
    c = new_cs + adj
    m_next = (i2 == (j2 + 1)).astype(jnp.float32)
    nxt = jnp.dot(c, m_next, precision=hi, preferred_element_type=jnp.float32)
    d = jnp.where(col <= _NUM_KNOTS - 2, nxt - c, 0.0)
    c_ref[...] = c
    d_ref[...] = d


def _project_tables(coefficients):
    return pl.pallas_call(
        _project_body,
        out_shape=[
            jax.ShapeDtypeStruct((_NUM_ACT, _NUM_KNOTS), jnp.float32),
            jax.ShapeDtypeStruct((_NUM_ACT, _NUM_KNOTS), jnp.float32),
        ],
    )(coefficients)


def _sc_body(x_hbm, c_hbm, d_hbm, out_hbm,
             in0, in1, out0, out1, crow, drow,
             sin0, sin1, sout0, sout1):
    wid = lax.axis_index("s") * _NC + lax.axis_index("c")
    inbufs = (in0, in1)
    outbufs = (out0, out1)
    sins = (sin0, sin1)
    souts = (sout0, sout1)

    def x_slice(k):
        img = wid * _IMGS_PER + k // _NCHUNK
        ck = lax.rem(k, _NCHUNK)
        return x_hbm.at[img, pl.ds(ck * _HCHUNK, _HCHUNK), :]

    def o_slice(k):
        img = wid * _IMGS_PER + k // _NCHUNK
        ck = lax.rem(k, _NCHUNK)
        return out_hbm.at[img, pl.ds(ck * _HCHUNK, _HCHUNK), :]

    pltpu.async_copy(x_slice(0), inbufs[0], sins[0])

    def do_chunk(k, carry):
        b = lax.rem(k, 2)
        ck = lax.rem(k, _NCHUNK)

        @pl.when(ck == 0)
        def _():
            img = wid * _IMGS_PER + k // _NCHUNK
            chan = lax.rem(img, _NUM_ACT)
            pltpu.sync_copy(c_hbm.at[chan], crow)
            pltpu.sync_copy(d_hbm.at[chan], drow)

        for bb in range(2):
            @pl.when(b == bb)
            def _():
                ib = inbufs[bb]
                ob = outbufs[bb]

                @pl.when(k + 1 < _NTOT)
                def _():
                    pltpu.async_copy(
                        x_slice(k + 1), inbufs[1 - bb], sins[1 - bb])

                pltpu.make_async_copy(x_slice(k), ib, sins[bb]).wait()
                @pl.when(k >= 2)
                def _():
                    pltpu.make_async_copy(
                        ob, o_slice(k - 2), souts[bb]).wait()

                @plsc.parallel_loop(0, _HCHUNK, step=1, unroll=1)
                def _compute(rr):
                    for u in range(_WVECS):
                        off = u * _L
                        xv = ib[rr, pl.ds(off, _L)]
                        t = (xv - jnp.float32(_X_MIN)) * jnp.float32(_INV_STEP)
                        tcl = jnp.minimum(jnp.maximum(t, 0.0),
                                          jnp.float32(_T_TOP))
                        idx = tcl.astype(jnp.int32)
                        fr = t - idx.astype(jnp.float32)
                        c0 = plsc.load_gather(crow, [idx])
                        dd = plsc.load_gather(drow, [idx])
                        ob[rr, pl.ds(off, _L)] = c0 + dd * fr

                pltpu.async_copy(ob, o_slice(k), souts[bb])
        return carry

    lax.fori_loop(0, _NTOT, do_chunk, 0)
    pltpu.make_async_copy(outbufs[0], o_slice(_NTOT - 2), souts[0]).wait()
    pltpu.make_async_copy(outbufs[1], o_slice(_NTOT - 1), souts[1]).wait()


@jax.jit
def kernel(x, coefficients):
    ctab, dtab = _project_tables(coefficients)
    xf = x.reshape(_IMGS, _H, _W)

    mesh = plsc.VectorSubcoreMesh(core_axis_name="c", subcore_axis_name="s")
    run = pl.kernel(
        _sc_body,
        out_type=jax.ShapeDtypeStruct((_IMGS, _H, _W), jnp.float32),
        mesh=mesh,
        compiler_params=pltpu.CompilerParams(needs_layout_passes=False),
        scratch_types=[
            pltpu.VMEM((_HCHUNK, _W), jnp.float32),
            pltpu.VMEM((_HCHUNK, _W), jnp.float32),
            pltpu.VMEM((_HCHUNK, _W), jnp.float32),
            pltpu.VMEM((_HCHUNK, _W), jnp.float32),
            pltpu.VMEM((_NUM_KNOTS,), jnp.float32),
            pltpu.VMEM((_NUM_KNOTS,), jnp.float32),
            pltpu.SemaphoreType.DMA,
            pltpu.SemaphoreType.DMA,
            pltpu.SemaphoreType.DMA,
            pltpu.SemaphoreType.DMA,
        ],
    )
    out = run(xf, ctab, dtab)
    return out.reshape(x.shape)


# speedup vs baseline: 3431.6103x; 1.0015x over previous
import numpy as np
import jax
import jax.numpy as jnp
from jax import lax
from jax.experimental import pallas as pl
from jax.experimental.pallas import tpu as pltpu
from jax.experimental.pallas import tpu_sc as plsc

_NUM_ACT = 96
_NUM_KNOTS = 64
_X_MIN = -4.0
_X_MAX = 4.0
_STEP = (_X_MAX - _X_MIN) / (_NUM_KNOTS - 1)
_INV_STEP = (_NUM_KNOTS - 1) / (_X_MAX - _X_MIN)
_XS_TOP = float(np.float32(np.float32(_X_MAX - _STEP) - np.float32(_X_MIN)))
_T_TOP = float(np.float32(np.float32(_XS_TOP) / np.float32(_STEP)))

_NC, _NS, _L = 2, 16, 16
_NW = _NC * _NS

_B, _C, _H, _W = 4, 96, 384, 384
_IMGS = _B * _C
_IMGS_PER = _IMGS // _NW
_HCHUNK = 64
_NCHUNK = _H // _HCHUNK
_WVECS = _W // _L
_NTOT = _IMGS_PER * _NCHUNK


def _project_body(cs_ref, c_ref, d_ref):
    cs = cs_ref[...]
    i2 = lax.broadcasted_iota(jnp.int32, (_NUM_KNOTS, _NUM_KNOTS), 0)
    j2 = lax.broadcasted_iota(jnp.int32, (_NUM_KNOTS, _NUM_KNOTS), 1)
    col = lax.broadcasted_iota(jnp.int32, (_NUM_ACT, _NUM_KNOTS), 1)

    hi = jax.lax.Precision.HIGHEST
    m_prev = (i2 == (j2 - 1)).astype(jnp.float32)
    prev = jnp.dot(cs, m_prev, precision=hi, preferred_element_type=jnp.float32)
    s = (cs - prev) * jnp.float32(1.0 / _STEP)
    s = jnp.where((col >= 2) & (col <= _NUM_KNOTS - 2), s, 0.0)
    tri = (i2 <= j2).astype(jnp.float32)
    new_cs = jnp.dot(s, tri, precision=hi, preferred_element_type=jnp.float32) * jnp.float32(_STEP)
    adj = jnp.mean(cs - new_cs, axis=1, keepdims=True)
    c = new_cs + adj
    m_next = (i2 == (j2 + 1)).astype(jnp.float32)
    nxt = jnp.dot(c, m_next, precision=hi, preferred_element_type=jnp.float32)
    d = jnp.where(col <= _NUM_KNOTS - 2, nxt - c, 0.0)
    c_ref[...] = c
    d_ref[...] = d


def _project_tables(coefficients):
    return pl.pallas_call(
        _project_body,
        out_shape=[
            jax.ShapeDtypeStruct((_NUM_ACT, _NUM_KNOTS), jnp.float32),
            jax.ShapeDtypeStruct((_NUM_ACT, _NUM_KNOTS), jnp.float32),
        ],
    )(coefficients)


def _sc_body(x_hbm, c_hbm, d_hbm, out_hbm,
             in0, in1, out0, out1, crow, drow,
             sin0, sin1, sout0, sout1):
    wid = lax.axis_index("s") * _NC + lax.axis_index("c")
    inbufs = (in0, in1)
    outbufs = (out0, out1)
    sins = (sin0, sin1)
    souts = (sout0, sout1)

    def x_slice(k):
        img = wid * _IMGS_PER + k // _NCHUNK
        ck = lax.rem(k, _NCHUNK)
        return x_hbm.at[img, pl.ds(ck * _HCHUNK, _HCHUNK), :]

    def o_slice(k):
        img = wid * _IMGS_PER + k // _NCHUNK
        ck = lax.rem(k, _NCHUNK)
        return out_hbm.at[img, pl.ds(ck * _HCHUNK, _HCHUNK), :]

    pltpu.async_copy(x_slice(0), inbufs[0], sins[0])

    def do_chunk(k, carry):
        b = lax.rem(k, 2)
        ck = lax.rem(k, _NCHUNK)

        @pl.when(ck == 0)
        def _():
            img = wid * _IMGS_PER + k // _NCHUNK
            chan = lax.rem(img, _NUM_ACT)
            pltpu.sync_copy(c_hbm.at[chan], crow)
            pltpu.sync_copy(d_hbm.at[chan], drow)

        for bb in range(2):
            @pl.when(b == bb)
            def _():
                ib = inbufs[bb]
                ob = outbufs[bb]

                @pl.when(k + 1 < _NTOT)
                def _():
                    pltpu.async_copy(
                        x_slice(k + 1), inbufs[1 - bb], sins[1 - bb])

                pltpu.make_async_copy(x_slice(k), ib, sins[bb]).wait()
                @pl.when(k >= 2)
                def _():
                    pltpu.make_async_copy(
                        ob, o_slice(k - 2), souts[bb]).wait()

                @plsc.parallel_loop(0, _HCHUNK, step=1, unroll=1)
                def _compute(rr):
                    for u in range(_WVECS):
                        off = u * _L
                        xv = ib[rr, pl.ds(off, _L)]
                        t = (xv - jnp.float32(_X_MIN)) * jnp.float32(_INV_STEP)
                        tcl = jnp.minimum(jnp.maximum(t, 0.0),
                                          jnp.float32(_T_TOP))
                        idx = tcl.astype(jnp.int32)
                        fr = t - idx.astype(jnp.float32)
                        c0 = plsc.load_gather(crow, [idx])
                        dd = plsc.load_gather(drow, [idx])
                        ob[rr, pl.ds(off, _L)] = c0 + dd * fr

                pltpu.async_copy(ob, o_slice(k), souts[bb])
        return carry

    lax.fori_loop(0, _NTOT, do_chunk, 0)
    pltpu.make_async_copy(outbufs[0], o_slice(_NTOT - 2), souts[0]).wait()
    pltpu.make_async_copy(outbufs[1], o_slice(_NTOT - 1), souts[1]).wait()


@jax.jit
def kernel(x, coefficients):
    ctab, dtab = _project_tables(coefficients)
    xf = x.reshape(_IMGS, _H, _W)

    mesh = plsc.VectorSubcoreMesh(core_axis_name="c", subcore_axis_name="s")
    run = pl.kernel(
        _sc_body,
        out_type=jax.ShapeDtypeStruct((_IMGS, _H, _W), jnp.float32),
        mesh=mesh,
        compiler_params=pltpu.CompilerParams(needs_layout_passes=False),
        scratch_types=[
            pltpu.VMEM((_HCHUNK, _W), jnp.float32),
            pltpu.VMEM((_HCHUNK, _W), jnp.float32),
            pltpu.VMEM((_HCHUNK, _W), jnp.float32),
            pltpu.VMEM((_HCHUNK, _W), jnp.float32),
            pltpu.VMEM((_NUM_KNOTS,), jnp.float32),
            pltpu.VMEM((_NUM_KNOTS,), jnp.float32),
            pltpu.SemaphoreType.DMA,
            pltpu.SemaphoreType.DMA,
            pltpu.SemaphoreType.DMA,
            pltpu.SemaphoreType.DMA,
        ],
    )
    out = run(xf, ctab, dtab)
    return out.reshape(x.shape)
